# Initial kernel scaffold; baseline (speedup 1.0000x reference)
import jax
import jax.numpy as jnp
from jax.experimental import pallas as pl


def kernel(points, mesh_values):
    raise NotImplementedError("write your pallas kernel here")


# speedup vs baseline: 7.9748x; 7.9748x over previous
import functools

import jax
import jax.numpy as jnp
from jax import lax
from jax.experimental import pallas as pl
from jax.experimental.pallas import tpu as pltpu
from jax.experimental.pallas import tpu_sc as plsc

N_MESH = 96
N_CHANNELS = 32
N_POINTS = 50000

NC = 2
NS = 16
LANES = 16
NW = NC * NS

B = 128
NB = 13
PTS_PER_W = B * NB
NPAD = NW * PTS_PER_W

_INV27 = [(a, b, c) for a in range(3) for b in range(3) for c in range(3)]


def _axis_cells_weights(p, spacing):
    pc = p / spacing
    rp_i = (pc + 0.5).astype(jnp.int32)
    rp_f = rp_i.astype(jnp.float32)
    d = pc - rp_f
    cm = lax.rem(rp_i + (N_MESH - 1), N_MESH)
    c0 = lax.rem(rp_i, N_MESH)
    cp = lax.rem(rp_i + 1, N_MESH)
    t = d + d
    wm = (t - 1.0) * (t - 1.0) * 0.125
    w0 = 0.75 - d * d
    wp = (t + 1.0) * (t + 1.0) * 0.125
    return (cm, c0, cp), (wm, w0, wp)


def _sc_body(table, pts, out, idx_v, w_v, pts_v, gath_v, out_v, sem):
    wid = lax.axis_index("s") * NC + lax.axis_index("c")
    tile_base = wid * PTS_PER_W
    spacing = jnp.float32(9.6 / N_MESH)

    def batch_body(b, carry):
        pbase = tile_base + b * B
        pltpu.sync_copy(pts.at[:, pl.ds(pbase, B)], pts_v)

        def group_body(g, carry2):
            sl = pl.ds(g * LANES, LANES)
            px = pts_v[0, sl]
            py = pts_v[1, sl]
            pz = pts_v[2, sl]
            xc, xw = _axis_cells_weights(px, spacing)
            yc, yw = _axis_cells_weights(py, spacing)
            zc, zw = _axis_cells_weights(pz, spacing)
            for s, (a, bb, c) in enumerate(_INV27):
                idx_v[s, sl] = (xc[a] * N_MESH + yc[bb]) * N_MESH + zc[c]
                w_v[s, sl] = xw[a] * yw[bb] * zw[c]
            return carry2

        lax.fori_loop(0, B // LANES, group_body, 0)

        handles = [
            pltpu.async_copy(table.at[idx_v.at[s]], gath_v.at[s], sem)
            for s in range(27)
        ]
        for h in handles:
            h.wait()

        def group_acc(g, carry3):
            base = g * LANES
            wrows = [w_v[s, pl.ds(base, LANES)] for s in range(27)]
            for lane in range(LANES):
                p = base + lane
                acc0 = jnp.zeros((LANES,), jnp.float32)
                acc1 = jnp.zeros((LANES,), jnp.float32)
                for s in range(27):
                    w = wrows[s][lane]
                    acc0 = acc0 + gath_v[s, p, pl.ds(0, LANES)] * w
                    acc1 = acc1 + gath_v[s, p, pl.ds(LANES, LANES)] * w
                out_v[p, pl.ds(0, LANES)] = acc0
                out_v[p, pl.ds(LANES, LANES)] = acc1
            return carry3

        lax.fori_loop(0, B // LANES, group_acc, 0)
        pltpu.sync_copy(out_v, out.at[pl.ds(pbase, B), :])
        return carry

    lax.fori_loop(0, NB, batch_body, 0)


_sc_interp = pl.kernel(
    _sc_body,
    out_type=jax.ShapeDtypeStruct((NPAD, N_CHANNELS), jnp.float32),
    mesh=plsc.VectorSubcoreMesh(core_axis_name="c", subcore_axis_name="s"),
    scratch_types=[
        pltpu.VMEM((27, B), jnp.int32),
        pltpu.VMEM((27, B), jnp.float32),
        pltpu.VMEM((3, B), jnp.float32),
        pltpu.VMEM((27, B, N_CHANNELS), jnp.float32),
        pltpu.VMEM((B, N_CHANNELS), jnp.float32),
        pltpu.SemaphoreType.DMA,
    ],
    compiler_params=pltpu.CompilerParams(use_tc_tiling_on_sc=False),
)


def kernel(points, mesh_values):
    table = jnp.transpose(mesh_values, (1, 2, 3, 0)).reshape(-1, N_CHANNELS)
    pts = jnp.pad(points.T, ((0, 0), (0, NPAD - N_POINTS)))
    out = _sc_interp(table, pts)
    return out[:N_POINTS]


# speedup vs baseline: 10.1008x; 1.2666x over previous
import jax
import jax.numpy as jnp
from jax import lax
from jax.experimental import pallas as pl
from jax.experimental.pallas import tpu as pltpu
from jax.experimental.pallas import tpu_sc as plsc

N_MESH = 96
N_CHANNELS = 32
N_POINTS = 50000

NC = 2
NS = 16
LANES = 16
NW = NC * NS

B = 64
NBATCH = 25
LEN_BIG = 1568
LEN_SMALL = 1560
N_BIG = 10

_INV27 = [(a, b, c) for a in range(3) for b in range(3) for c in range(3)]


def _axis_cells_weights(p, spacing):
    pc = p / spacing
    rp_i = (pc + 0.5).astype(jnp.int32)
    rp_f = rp_i.astype(jnp.float32)
    d = pc - rp_f
    cm = lax.rem(rp_i + (N_MESH - 1), N_MESH)
    c0 = lax.rem(rp_i, N_MESH)
    cp = lax.rem(rp_i + 1, N_MESH)
    t = d + d
    wm = (t - 1.0) * (t - 1.0) * 0.125
    w0 = 0.75 - d * d
    wp = (t + 1.0) * (t + 1.0) * 0.125
    return (cm, c0, cp), (wm, w0, wp)


def _sc_body(table, pts, out,
             idx0, idx1, w0, w1, pts0, pts1, gath0, gath1, out_v,
             sem0, sem1):
    wid = lax.axis_index("s") * NC + lax.axis_index("c")
    start = wid * LEN_SMALL + 8 * jnp.minimum(wid, N_BIG)
    length = jnp.where(wid < N_BIG, LEN_BIG, LEN_SMALL)
    last_off = length - B
    spacing = jnp.float32(9.6 / N_MESH)

    def pbase_of(b):
        return start + jnp.minimum(b * B, last_off)

    def stage(idx_r, w_r, pts_r, gath_r, sem, b):
        pltpu.sync_copy(pts.at[:, pl.ds(pbase_of(b), B)], pts_r)

        def group_body(g, carry):
            sl = pl.ds(g * LANES, LANES)
            xc, xw = _axis_cells_weights(pts_r[0, sl], spacing)
            yc, yw = _axis_cells_weights(pts_r[1, sl], spacing)
            zc, zw = _axis_cells_weights(pts_r[2, sl], spacing)
            for s, (a, bb, c) in enumerate(_INV27):
                idx_r[s, sl] = (xc[a] * N_MESH + yc[bb]) * N_MESH + zc[c]
                w_r[s, sl] = xw[a] * yw[bb] * zw[c]
            return carry

        lax.fori_loop(0, B // LANES, group_body, 0)
        for s in range(27):
            pltpu.async_copy(table.at[idx_r.at[s]], gath_r.at[s], sem)

    def drain(idx_r, gath_r, sem):
        for s in range(27):
            pltpu.make_async_copy(table.at[idx_r.at[s]], gath_r.at[s], sem).wait()

    def accumulate(w_r, gath_r, b):
        def group_acc(g, carry):
            base = g * LANES
            wrows = [w_r[s, pl.ds(base, LANES)] for s in range(27)]

            def lane_body(l, carry2):
                lv = jnp.full((LANES,), l, jnp.int32)
                p = base + l
                acc0 = jnp.zeros((LANES,), jnp.float32)
                acc1 = jnp.zeros((LANES,), jnp.float32)
                for s in range(27):
                    w16 = wrows[s].at[lv].get(mode="promise_in_bounds")
                    acc0 = acc0 + gath_r[s, p, pl.ds(0, LANES)] * w16
                    acc1 = acc1 + gath_r[s, p, pl.ds(LANES, LANES)] * w16
                out_v[p, pl.ds(0, LANES)] = acc0
                out_v[p, pl.ds(LANES, LANES)] = acc1
                return carry2

            lax.fori_loop(0, LANES, lane_body, 0)
            return carry

        lax.fori_loop(0, B // LANES, group_acc, 0)
        pltpu.sync_copy(out_v, out.at[pl.ds(pbase_of(b), B), :])

    slot0 = (idx0, w0, pts0, gath0, sem0)
    slot1 = (idx1, w1, pts1, gath1, sem1)

    def stage_slot(slot, b):
        idx_r, w_r, pts_r, gath_r, sem = slot
        stage(idx_r, w_r, pts_r, gath_r, sem, b)

    def process_slot(slot, b):
        idx_r, w_r, pts_r, gath_r, sem = slot
        drain(idx_r, gath_r, sem)
        accumulate(w_r, gath_r, b)

    stage_slot(slot0, jnp.int32(0))

    def outer(i, carry):
        b0 = 2 * i
        b1 = 2 * i + 1

        @pl.when(b0 + 1 < NBATCH)
        def _():
            stage_slot(slot1, b0 + 1)

        process_slot(slot0, b0)

        @pl.when(b1 + 1 < NBATCH)
        def _():
            stage_slot(slot0, b1 + 1)

        @pl.when(b1 < NBATCH)
        def _():
            process_slot(slot1, b1)

        return carry

    lax.fori_loop(0, (NBATCH + 1) // 2, outer, 0)


_sc_interp = pl.kernel(
    _sc_body,
    out_type=jax.ShapeDtypeStruct((N_POINTS, N_CHANNELS), jnp.float32),
    mesh=plsc.VectorSubcoreMesh(core_axis_name="c", subcore_axis_name="s"),
    scratch_types=[
        pltpu.VMEM((27, B), jnp.int32),
        pltpu.VMEM((27, B), jnp.int32),
        pltpu.VMEM((27, B), jnp.float32),
        pltpu.VMEM((27, B), jnp.float32),
        pltpu.VMEM((3, B), jnp.float32),
        pltpu.VMEM((3, B), jnp.float32),
        pltpu.VMEM((27, B, N_CHANNELS), jnp.float32),
        pltpu.VMEM((27, B, N_CHANNELS), jnp.float32),
        pltpu.VMEM((B, N_CHANNELS), jnp.float32),
        pltpu.SemaphoreType.DMA,
        pltpu.SemaphoreType.DMA,
    ],
    compiler_params=pltpu.CompilerParams(use_tc_tiling_on_sc=False),
)


def kernel(points, mesh_values):
    table = jnp.transpose(mesh_values, (1, 2, 3, 0)).reshape(-1, N_CHANNELS)
    pts = points.T
    return _sc_interp(table, pts)
